# baseline (speedup 1.0000x reference)
import jax
import jax.numpy as jnp
from jax.experimental import pallas as pl
from jax.experimental.pallas import tpu as pltpu

B, H, D = 64, 32, 64
NUM_SLOTS = 128


def _slot_kernel(inv_ref, cache_ref, q_ref, k_ref, v_ref, slope_ref,
                 newc_ref, out_ref):
    s = pl.program_id(0)
    b = inv_ref[s]
    active = b >= 0
    kv_old = cache_ref[0]
    k3 = k_ref[0, :, 0, :]
    v3 = v_ref[0, :, 0, :]
    q3 = q_ref[0, :, 0, :]
    ratio = jnp.exp(-slope_ref[0])
    kv_new = k3[:, :, None] * v3[:, None, :] + ratio[:, None, None] * kv_old
    newc_ref[0] = jnp.where(active, kv_new, kv_old)
    out_ref[0, :, 0, :] = jnp.sum(q3[:, :, None] * kv_new, axis=1)


def kernel(q, k, v, kv_caches, slope_rate, slot_idx):
    slot_idx = slot_idx.astype(jnp.int32)
    inv = jnp.full((NUM_SLOTS,), -1, jnp.int32).at[slot_idx].set(
        jnp.arange(B, dtype=jnp.int32))
    slope2 = slope_rate.reshape(1, H)

    grid_spec = pltpu.PrefetchScalarGridSpec(
        num_scalar_prefetch=1,
        grid=(NUM_SLOTS,),
        in_specs=[
            pl.BlockSpec((1, H, D, D), lambda s, inv: (s, 0, 0, 0)),
            pl.BlockSpec((1, H, 1, D),
                         lambda s, inv: (jnp.maximum(inv[s], 0), 0, 0, 0)),
            pl.BlockSpec((1, H, 1, D),
                         lambda s, inv: (jnp.maximum(inv[s], 0), 0, 0, 0)),
            pl.BlockSpec((1, H, 1, D),
                         lambda s, inv: (jnp.maximum(inv[s], 0), 0, 0, 0)),
            pl.BlockSpec((1, H), lambda s, inv: (0, 0)),
        ],
        out_specs=[
            pl.BlockSpec((1, H, D, D), lambda s, inv: (s, 0, 0, 0)),
            pl.BlockSpec((1, H, 1, D), lambda s, inv: (s, 0, 0, 0)),
        ],
    )
    new_cache, out_s = pl.pallas_call(
        _slot_kernel,
        grid_spec=grid_spec,
        out_shape=[
            jax.ShapeDtypeStruct((NUM_SLOTS, H, D, D), jnp.float32),
            jax.ShapeDtypeStruct((NUM_SLOTS, H, 1, D), jnp.float32),
        ],
    )(inv, kv_caches, q, k, v, slope2)
    output = jnp.take(out_s, slot_idx, axis=0)
    return output, new_cache


# speedup vs baseline: 1.0334x; 1.0334x over previous
import jax
import jax.numpy as jnp
from jax.experimental import pallas as pl
from jax.experimental.pallas import tpu as pltpu

B, H, D = 64, 32, 64
NUM_SLOTS = 128


def _slot_kernel(inv_ref, cache_ref, q_ref, k_ref, v_ref, slope_ref,
                 newc_ref, out_ref):
    s = pl.program_id(0)
    b = inv_ref[s]
    active = b >= 0
    bc = jnp.maximum(b, 0)
    kv_old = cache_ref[0]

    @pl.when(active)
    def _update():
        k3 = k_ref[bc, :, 0, :]
        v3 = v_ref[bc, :, 0, :]
        q3 = q_ref[bc, :, 0, :]
        ratio = jnp.exp(-slope_ref[0])
        kv_new = (k3[:, :, None] * v3[:, None, :]
                  + ratio[:, None, None] * kv_old)
        newc_ref[0] = kv_new
        out_ref[s, :, 0, :] = jnp.sum(q3[:, :, None] * kv_new, axis=1)

    @pl.when(jnp.logical_not(active))
    def _copy():
        newc_ref[0] = kv_old


def kernel(q, k, v, kv_caches, slope_rate, slot_idx):
    slot_idx = slot_idx.astype(jnp.int32)
    inv = jnp.full((NUM_SLOTS,), -1, jnp.int32).at[slot_idx].set(
        jnp.arange(B, dtype=jnp.int32))
    slope2 = slope_rate.reshape(1, H)

    grid_spec = pltpu.PrefetchScalarGridSpec(
        num_scalar_prefetch=1,
        grid=(NUM_SLOTS,),
        in_specs=[
            pl.BlockSpec((1, H, D, D), lambda s, inv: (s, 0, 0, 0)),
            pl.BlockSpec((B, H, 1, D), lambda s, inv: (0, 0, 0, 0)),
            pl.BlockSpec((B, H, 1, D), lambda s, inv: (0, 0, 0, 0)),
            pl.BlockSpec((B, H, 1, D), lambda s, inv: (0, 0, 0, 0)),
            pl.BlockSpec((1, H), lambda s, inv: (0, 0)),
        ],
        out_specs=[
            pl.BlockSpec((1, H, D, D), lambda s, inv: (s, 0, 0, 0)),
            pl.BlockSpec((NUM_SLOTS, H, 1, D), lambda s, inv: (0, 0, 0, 0)),
        ],
    )
    new_cache, out_s = pl.pallas_call(
        _slot_kernel,
        grid_spec=grid_spec,
        out_shape=[
            jax.ShapeDtypeStruct((NUM_SLOTS, H, D, D), jnp.float32),
            jax.ShapeDtypeStruct((NUM_SLOTS, H, 1, D), jnp.float32),
        ],
    )(inv, kv_caches, q, k, v, slope2)
    output = jnp.take(out_s, slot_idx, axis=0)
    return output, new_cache
